# baseline (speedup 1.0000x reference)
import functools
import math

import jax
import jax.numpy as jnp
from jax import lax
from jax.experimental import pallas as pl
from jax.experimental.pallas import tpu as pltpu
from jax.experimental.pallas import tpu_sc as plsc

H, DH, KN = 8, 64, 32
THETA = 10000.0
REL_DIST_CUTOFF = 5000.0
REL_DIST_SCALE = 100.0
EPS = 1e-8
SCALE = DH ** -0.5

R_TOPK = 256
R_ATT = 64
SC_CHUNK = 64


def _qkv_body(x_ref, w_ref, c_ref, q_ref, kvb_ref, c128_ref):
    y = jnp.dot(x_ref[...], w_ref[...], preferred_element_type=jnp.float32)
    q_ref[...] = y[:, :512]
    kb = y[:, 512:1024].astype(jnp.bfloat16).astype(jnp.float32)
    vb = y[:, 1024:].astype(jnp.bfloat16).astype(jnp.float32)
    ku = lax.shift_right_logical(lax.bitcast_convert_type(kb, jnp.uint32),
                                 jnp.uint32(16))
    vu = lax.bitcast_convert_type(vb, jnp.uint32) & jnp.uint32(0xFFFF0000)
    kvb_ref[...] = lax.bitcast_convert_type(ku | vu, jnp.int32)
    c128_ref[...] = jnp.pad(c_ref[...], ((0, 0), (0, 112)))


def _qkv(x, w, c16):
    bn, d = x.shape
    r = 256
    return pl.pallas_call(
        _qkv_body,
        grid=(bn // r,),
        in_specs=[
            pl.BlockSpec((r, d), lambda i: (i, 0)),
            pl.BlockSpec((d, 3 * 512), lambda i: (0, 0)),
            pl.BlockSpec((r, 16), lambda i: (i, 0)),
        ],
        out_specs=[
            pl.BlockSpec((r, 512), lambda i: (i, 0)),
            pl.BlockSpec((r, 512), lambda i: (i, 0)),
            pl.BlockSpec((r, 128), lambda i: (i, 0)),
        ],
        out_shape=[
            jax.ShapeDtypeStruct((bn, 512), jnp.float32),
            jax.ShapeDtypeStruct((bn, 512), jnp.int32),
            jax.ShapeDtypeStruct((bn, 128), jnp.float32),
        ],
    )(x, w, c16)


def _topk_body(n, xi_ref, xt_ref, idx_ref):
    b = pl.program_id(0)
    xi = xi_ref[0]
    xt = xt_ref[0]
    dsq = (xi[:, 0:1] - xt[0:1, :]) ** 2
    dsq += (xi[:, 1:2] - xt[1:2, :]) ** 2
    dsq += (xi[:, 2:3] - xt[2:3, :]) ** 2
    r = xi.shape[0]
    iota = lax.broadcasted_iota(jnp.int32, (r, n), 1)
    big = jnp.int32(n)
    inf = jnp.float32(jnp.inf)
    base = b * n
    for it in range(KN):
        m = jnp.min(dsq, axis=1, keepdims=True)
        tie = dsq == m
        a = jnp.min(jnp.where(tie, iota, big), axis=1, keepdims=True)
        idx_ref[0, :, it:it + 1] = a + base
        dsq = jnp.where(tie, inf, dsq)


def _topk(coors, coors_t):
    b, n, _ = coors.shape
    return pl.pallas_call(
        functools.partial(_topk_body, n),
        grid=(b, n // R_TOPK),
        in_specs=[
            pl.BlockSpec((1, R_TOPK, 3), lambda bi, i: (bi, i, 0)),
            pl.BlockSpec((1, 3, n), lambda bi, i: (bi, 0, 0)),
        ],
        out_specs=pl.BlockSpec((1, R_TOPK, KN), lambda bi, i: (bi, i, 0)),
        out_shape=jax.ShapeDtypeStruct((b, n, KN), jnp.int32),
    )(coors, coors_t)


def _sc_gather(kvb, c128, idx_flat):
    g = idx_flat.shape[0]
    ch = SC_CHUNK
    info = plsc.get_sparse_core_info()
    nw = info.num_cores * info.num_subcores
    per_w = g // nw
    n_chunks = per_w // ch
    mesh = plsc.VectorSubcoreMesh(core_axis_name="c", subcore_axis_name="s")

    @functools.partial(
        pl.kernel,
        mesh=mesh,
        out_type=[
            jax.ShapeDtypeStruct((g, 512), jnp.int32),
            jax.ShapeDtypeStruct((g, 128), jnp.float32),
        ],
        scratch_types=[
            pltpu.VMEM((ch,), jnp.int32),
            pltpu.VMEM((ch,), jnp.int32),
            pltpu.VMEM((ch, 512), jnp.int32),
            pltpu.VMEM((ch, 512), jnp.int32),
            pltpu.VMEM((ch, 128), jnp.float32),
            pltpu.VMEM((ch, 128), jnp.float32),
            pltpu.SemaphoreType.DMA,
            pltpu.SemaphoreType.DMA,
            pltpu.SemaphoreType.DMA,
            pltpu.SemaphoreType.DMA,
        ],
    )
    def k(kvb_hbm, c128_hbm, idx_hbm, kvg_hbm, cg_hbm,
          idx_a, idx_b, kv_a, kv_b, c_a, c_b, semka, semca, semkb, semcb):
        wid = lax.axis_index("s") * info.num_cores + lax.axis_index("c")
        wbase = wid * per_w
        bufs = ((idx_a, kv_a, c_a, semka, semca),
                (idx_b, kv_b, c_b, semkb, semcb))

        def fire(ci, bf):
            idx_v, kv_v, c_v, semk, semc = bf
            off = pl.multiple_of(wbase + ci * ch, ch)
            pltpu.sync_copy(idx_hbm.at[pl.ds(off, ch)], idx_v)
            pltpu.async_copy(kvb_hbm.at[idx_v], kv_v, semk)
            pltpu.async_copy(c128_hbm.at[idx_v], c_v, semc)

        def drain_store(ci, bf):
            idx_v, kv_v, c_v, semk, semc = bf
            off = pl.multiple_of(wbase + ci * ch, ch)
            pltpu.make_async_copy(kvb_hbm.at[pl.ds(0, ch)], kv_v, semk).wait()
            pltpu.make_async_copy(c128_hbm.at[pl.ds(0, ch)], c_v, semc).wait()
            pltpu.sync_copy(kv_v, kvg_hbm.at[pl.ds(off, ch)])
            pltpu.sync_copy(c_v, cg_hbm.at[pl.ds(off, ch)])

        fire(0, bufs[0])

        def body(p, carry):
            e = p * 2
            fire(e + 1, bufs[1])
            drain_store(e, bufs[0])

            @pl.when(e + 2 < n_chunks)
            def _():
                fire(e + 2, bufs[0])

            drain_store(e + 1, bufs[1])
            return carry

        lax.fori_loop(0, n_chunks // 2, body, 0)

    return k(kvb, c128, idx_flat)


def _attn_body(q_ref, kvg_ref, cg_ref, xi_ref, wout_ref, bout_ref,
               cmw1_ref, cmb1_ref, cmw2_ref, cmb2_ref, cgw_ref, cgb_ref,
               cs_ref, cc_ref, out_ref, co_ref):
    r = q_ref.shape[0]
    x = r * KN
    f32 = jnp.float32

    w = lax.bitcast_convert_type(kvg_ref[...], jnp.uint32)
    kg = lax.bitcast_convert_type(
        lax.shift_left(w, jnp.full(w.shape, 16, jnp.uint32)), f32)
    vg = lax.bitcast_convert_type(w & jnp.uint32(0xFFFF0000), f32)
    cg = cg_ref[:, :16]

    xi16 = xi_ref[...]
    xi_rep = jnp.broadcast_to(xi16[:, None, :], (r, KN, 16)).reshape(x, 16)
    rel = xi_rep - cg
    d2 = jnp.sum(rel * rel, axis=1, keepdims=True) + 1e-30
    dist = jnp.sqrt(d2)

    t = jnp.minimum(dist * REL_DIST_SCALE, REL_DIST_CUTOFF)
    g32 = lax.broadcasted_iota(jnp.int32, (1, DH // 2), 1)
    inv_f = jnp.exp(g32.astype(f32) * (-math.log(THETA) * 2.0 / DH))
    ang = t * inv_f
    c32 = jnp.cos(ang)
    s32 = jnp.sin(ang)
    gi = lax.broadcasted_iota(jnp.int32, (DH // 2, 512), 0)
    cj = lax.broadcasted_iota(jnp.int32, (DH // 2, 512), 1)
    exp3 = ((cj % DH) // 2 == gi).astype(f32)
    cosf = jnp.dot(c32, exp3, preferred_element_type=f32)
    sinf = jnp.dot(s32, exp3, preferred_element_type=f32)
    kl = jnp.concatenate([kg[:, 1:], kg[:, :1]], axis=1)
    kr = jnp.concatenate([kg[:, 511:], kg[:, :511]], axis=1)
    even = (lax.broadcasted_iota(jnp.int32, (1, 512), 1) % 2) == 0
    krot = jnp.where(even, -kl, kr)
    k_rot = kg * cosf + krot * sinf

    q = q_ref[...]
    q_rep = jnp.broadcast_to(q[:, None, :], (r, KN, 512)).reshape(x, 512)
    ri = lax.broadcasted_iota(jnp.int32, (512, H), 0)
    ci = lax.broadcasted_iota(jnp.int32, (512, H), 1)
    sumblk = (ri // DH == ci).astype(f32)
    qk = jnp.dot(q_rep * k_rot, sumblk, preferred_element_type=f32) * SCALE

    sri = lax.broadcasted_iota(jnp.int32, (r, x), 0)
    sci = lax.broadcasted_iota(jnp.int32, (r, x), 1)
    seg = (sci // KN == sri).astype(f32)

    e = jnp.exp(qk)
    den = jnp.dot(seg, e, preferred_element_type=f32)
    den_rep = jnp.broadcast_to(den[:, None, :], (r, KN, H)).reshape(x, H)
    attn = e / den_rep

    expand = (ri.T // DH == ci.T).astype(f32)
    wexp = jnp.dot(attn, expand, preferred_element_type=f32)
    outt = jnp.dot(seg, wexp * vg, preferred_element_type=f32)
    out_ref[...] = jnp.dot(outt, wout_ref[...], preferred_element_type=f32) + bout_ref[...]

    hin = jnp.dot(qk, cmw1_ref[...], preferred_element_type=f32) + cmb1_ref[...]
    hid = 0.5 * hin * (1.0 + lax.erf(hin * (2.0 ** -0.5)))
    cw = jnp.dot(hid, cmw2_ref[...], preferred_element_type=f32) + cmb2_ref[...]
    gate = jnp.tanh(jnp.dot(qk, cgw_ref[...], preferred_element_type=f32) + cgb_ref[...])
    reln = rel / jnp.maximum(dist, EPS) * cs_ref[0, 0]
    s = jnp.sum(cw * gate * cc_ref[...], axis=1, keepdims=True)
    delta = jnp.dot(seg, s * reln, preferred_element_type=f32)
    co_ref[...] = xi16[:, :3] + delta[:, :3]


def _attn(q, kvg, cgath, c16, w_out, b_out, cm_w1, cm_b1, cm_w2, cm_b2,
          cg_w, cg_b, cs, cc):
    bn = q.shape[0]
    r = R_ATT
    x = r * KN
    const = lambda shape: pl.BlockSpec(shape, lambda i: tuple(0 for _ in shape))
    return pl.pallas_call(
        _attn_body,
        grid=(bn // r,),
        in_specs=[
            pl.BlockSpec((r, 512), lambda i: (i, 0)),
            pl.BlockSpec((x, 512), lambda i: (i, 0)),
            pl.BlockSpec((x, 128), lambda i: (i, 0)),
            pl.BlockSpec((r, 16), lambda i: (i, 0)),
            const((512, 512)),
            const((1, 512)),
            const((H, 16)),
            const((1, 16)),
            const((16, H)),
            const((1, H)),
            const((H, H)),
            const((1, H)),
            const((1, 1)),
            const((1, H)),
        ],
        out_specs=[
            pl.BlockSpec((r, 512), lambda i: (i, 0)),
            pl.BlockSpec((r, 3), lambda i: (i, 0)),
        ],
        out_shape=[
            jax.ShapeDtypeStruct((bn, 512), jnp.float32),
            jax.ShapeDtypeStruct((bn, 3), jnp.float32),
        ],
    )(q, kvg, cgath, c16, w_out, b_out, cm_w1, cm_b1, cm_w2, cm_b2, cg_w, cg_b, cs, cc)


def kernel(feats, coors, W_qkv, W_out, b_out, cm_W1, cm_b1, cm_W2, cm_b2,
           cg_W, cg_b, coors_scale, coors_combine):
    b, n, d = feats.shape
    bn = b * n

    c16 = jnp.pad(coors.reshape(bn, 3), ((0, 0), (0, 13)))
    q, kvb, c128 = _qkv(feats.reshape(bn, d), W_qkv, c16)
    idx = _topk(coors, coors.transpose(0, 2, 1))
    kvg, cgath = _sc_gather(kvb, c128, idx.reshape(bn * KN))
    out, coors_out = _attn(
        q, kvg, cgath, c16, W_out, b_out.reshape(1, 512),
        cm_W1, cm_b1.reshape(1, 16), cm_W2, cm_b2.reshape(1, H),
        cg_W, cg_b.reshape(1, H), coors_scale.reshape(1, 1),
        coors_combine.reshape(1, H))
    return out.reshape(b, n, d), coors_out.reshape(b, n, 3)


# speedup vs baseline: 1.1043x; 1.1043x over previous
import functools
import math

import jax
import jax.numpy as jnp
from jax import lax
from jax.experimental import pallas as pl
from jax.experimental.pallas import tpu as pltpu
from jax.experimental.pallas import tpu_sc as plsc

H, DH, KN = 8, 64, 32
THETA = 10000.0
REL_DIST_CUTOFF = 5000.0
REL_DIST_SCALE = 100.0
EPS = 1e-8
SCALE = DH ** -0.5

R_TOPK = 256
R_ATT = 64
SC_CHUNK = 64


def _qkv_body(x_ref, w_ref, c_ref, q_ref, kvb_ref, c128_ref):
    y = jnp.dot(x_ref[...], w_ref[...], preferred_element_type=jnp.float32)
    q_ref[...] = y[:, :512]
    kb = y[:, 512:1024].astype(jnp.bfloat16).astype(jnp.float32)
    vb = y[:, 1024:].astype(jnp.bfloat16).astype(jnp.float32)
    ku = lax.shift_right_logical(lax.bitcast_convert_type(kb, jnp.uint32),
                                 jnp.uint32(16))
    vu = lax.bitcast_convert_type(vb, jnp.uint32) & jnp.uint32(0xFFFF0000)
    kvb_ref[...] = lax.bitcast_convert_type(ku | vu, jnp.int32)
    c128_ref[...] = jnp.pad(c_ref[...], ((0, 0), (0, 112)))


def _qkv(x, w, c16):
    bn, d = x.shape
    r = 256
    return pl.pallas_call(
        _qkv_body,
        grid=(bn // r,),
        in_specs=[
            pl.BlockSpec((r, d), lambda i: (i, 0)),
            pl.BlockSpec((d, 3 * 512), lambda i: (0, 0)),
            pl.BlockSpec((r, 16), lambda i: (i, 0)),
        ],
        out_specs=[
            pl.BlockSpec((r, 512), lambda i: (i, 0)),
            pl.BlockSpec((r, 512), lambda i: (i, 0)),
            pl.BlockSpec((r, 128), lambda i: (i, 0)),
        ],
        out_shape=[
            jax.ShapeDtypeStruct((bn, 512), jnp.float32),
            jax.ShapeDtypeStruct((bn, 512), jnp.int32),
            jax.ShapeDtypeStruct((bn, 128), jnp.float32),
        ],
    )(x, w, c16)


def _topk_body(n, xi_ref, xt_ref, idx_ref):
    b = pl.program_id(0)
    xi = xi_ref[0]
    xt = xt_ref[0]
    dsq = (xi[:, 0:1] - xt[0:1, :]) ** 2
    dsq += (xi[:, 1:2] - xt[1:2, :]) ** 2
    dsq += (xi[:, 2:3] - xt[2:3, :]) ** 2
    r = xi.shape[0]
    iota = lax.broadcasted_iota(jnp.int32, (r, n), 1)
    big = jnp.int32(n)
    inf = jnp.float32(jnp.inf)
    base = b * n
    for it in range(KN):
        m = jnp.min(dsq, axis=1, keepdims=True)
        tie = dsq == m
        a = jnp.min(jnp.where(tie, iota, big), axis=1, keepdims=True)
        idx_ref[0, :, it:it + 1] = a + base
        dsq = jnp.where(tie, inf, dsq)


def _topk(coors, coors_t):
    b, n, _ = coors.shape
    return pl.pallas_call(
        functools.partial(_topk_body, n),
        grid=(b, n // R_TOPK),
        in_specs=[
            pl.BlockSpec((1, R_TOPK, 3), lambda bi, i: (bi, i, 0)),
            pl.BlockSpec((1, 3, n), lambda bi, i: (bi, 0, 0)),
        ],
        out_specs=pl.BlockSpec((1, R_TOPK, KN), lambda bi, i: (bi, i, 0)),
        out_shape=jax.ShapeDtypeStruct((b, n, KN), jnp.int32),
    )(coors, coors_t)


def _sc_gather(kvb, c128, idx_flat):
    g = idx_flat.shape[0]
    ch = SC_CHUNK
    info = plsc.get_sparse_core_info()
    nw = info.num_cores * info.num_subcores
    per_w = g // nw
    n_chunks = per_w // ch
    mesh = plsc.VectorSubcoreMesh(core_axis_name="c", subcore_axis_name="s")

    @functools.partial(
        pl.kernel,
        mesh=mesh,
        out_type=[
            jax.ShapeDtypeStruct((g, 512), jnp.int32),
            jax.ShapeDtypeStruct((g, 128), jnp.float32),
        ],
        scratch_types=[
            pltpu.VMEM((ch,), jnp.int32),
            pltpu.VMEM((ch,), jnp.int32),
            pltpu.VMEM((ch, 512), jnp.int32),
            pltpu.VMEM((ch, 512), jnp.int32),
            pltpu.VMEM((ch, 128), jnp.float32),
            pltpu.VMEM((ch, 128), jnp.float32),
            pltpu.SemaphoreType.DMA,
            pltpu.SemaphoreType.DMA,
            pltpu.SemaphoreType.DMA,
            pltpu.SemaphoreType.DMA,
        ],
    )
    def k(kvb_hbm, c128_hbm, idx_hbm, kvg_hbm, cg_hbm,
          idx_a, idx_b, kv_a, kv_b, c_a, c_b, semka, semca, semkb, semcb):
        wid = lax.axis_index("s") * info.num_cores + lax.axis_index("c")
        wbase = wid * per_w
        bufs = ((idx_a, kv_a, c_a, semka, semca),
                (idx_b, kv_b, c_b, semkb, semcb))

        def fire(ci, bf):
            idx_v, kv_v, c_v, semk, semc = bf
            off = pl.multiple_of(wbase + ci * ch, ch)
            pltpu.sync_copy(idx_hbm.at[pl.ds(off, ch)], idx_v)
            pltpu.async_copy(kvb_hbm.at[idx_v], kv_v, semk)
            pltpu.async_copy(c128_hbm.at[idx_v], c_v, semc)

        def drain_store(ci, bf):
            idx_v, kv_v, c_v, semk, semc = bf
            off = pl.multiple_of(wbase + ci * ch, ch)
            pltpu.make_async_copy(kvb_hbm.at[pl.ds(0, ch)], kv_v, semk).wait()
            pltpu.make_async_copy(c128_hbm.at[pl.ds(0, ch)], c_v, semc).wait()
            pltpu.sync_copy(kv_v, kvg_hbm.at[pl.ds(off, ch)])
            pltpu.sync_copy(c_v, cg_hbm.at[pl.ds(off, ch)])

        fire(0, bufs[0])

        def body(p, carry):
            e = p * 2
            fire(e + 1, bufs[1])
            drain_store(e, bufs[0])

            @pl.when(e + 2 < n_chunks)
            def _():
                fire(e + 2, bufs[0])

            drain_store(e + 1, bufs[1])
            return carry

        lax.fori_loop(0, n_chunks // 2, body, 0)

    return k(kvb, c128, idx_flat)


def _attn_body(q_ref, kvg_ref, cg_ref, xi_ref, wout_ref, bout_ref,
               cmw1_ref, cmb1_ref, cmw2_ref, cmb2_ref, cgw_ref, cgb_ref,
               cs_ref, cc_ref, out_ref, co_ref):
    r = q_ref.shape[0]
    x = r * KN
    f32 = jnp.float32

    w = lax.bitcast_convert_type(kvg_ref[...], jnp.uint32)
    kg = lax.bitcast_convert_type(
        lax.shift_left(w, jnp.full(w.shape, 16, jnp.uint32)), f32)
    vg = lax.bitcast_convert_type(w & jnp.uint32(0xFFFF0000), f32)
    cg = cg_ref[:, :16]

    xi16 = xi_ref[...]
    xi_rep = jnp.broadcast_to(xi16[:, None, :], (r, KN, 16)).reshape(x, 16)
    rel = xi_rep - cg
    d2 = jnp.sum(rel * rel, axis=1, keepdims=True) + 1e-30
    dist = jnp.sqrt(d2)

    t = jnp.minimum(dist * REL_DIST_SCALE, REL_DIST_CUTOFF)
    pair = lax.broadcasted_iota(jnp.int32, (1, DH), 1) // 2
    inv_f = jnp.exp(pair.astype(f32) * (-math.log(THETA) * 2.0 / DH))
    ang = t * inv_f
    q = lax.round(ang * (1.0 / math.pi),
                  lax.RoundingMethod.TO_NEAREST_EVEN)
    rt = (ang - q * 3.140625) - q * 9.67653589793e-4
    z = rt * rt
    cosp = 1.0 + z * (-0.5 + z * (1.0 / 24 + z * (-1.0 / 720 + z * (
        1.0 / 40320 + z * (-1.0 / 3628800)))))
    sinp = rt * (1.0 + z * (-1.0 / 6 + z * (1.0 / 120 + z * (
        -1.0 / 5040 + z * (1.0 / 362880)))))
    sgn = lax.shift_left(q.astype(jnp.int32), jnp.full(ang.shape, 31, jnp.int32))
    c64 = lax.bitcast_convert_type(
        lax.bitcast_convert_type(cosp, jnp.int32) ^ sgn, f32)
    s64 = lax.bitcast_convert_type(
        lax.bitcast_convert_type(sinp, jnp.int32) ^ sgn, f32)
    cosf = jnp.concatenate([c64] * H, axis=1)
    sinf = jnp.concatenate([s64] * H, axis=1)
    kl = jnp.concatenate([kg[:, 1:], kg[:, :1]], axis=1)
    kr = jnp.concatenate([kg[:, 511:], kg[:, :511]], axis=1)
    even = (lax.broadcasted_iota(jnp.int32, (1, 512), 1) % 2) == 0
    krot = jnp.where(even, -kl, kr)
    k_rot = kg * cosf + krot * sinf

    q = q_ref[...]
    q_rep = jnp.broadcast_to(q[:, None, :], (r, KN, 512)).reshape(x, 512)
    ri = lax.broadcasted_iota(jnp.int32, (512, H), 0)
    ci = lax.broadcasted_iota(jnp.int32, (512, H), 1)
    sumblk = (ri // DH == ci).astype(f32)
    qk = jnp.dot(q_rep * k_rot, sumblk, preferred_element_type=f32) * SCALE

    sri = lax.broadcasted_iota(jnp.int32, (r, x), 0)
    sci = lax.broadcasted_iota(jnp.int32, (r, x), 1)
    seg = (sci // KN == sri).astype(f32)

    e = jnp.exp(qk)
    den = jnp.dot(seg, e, preferred_element_type=f32)
    den_rep = jnp.broadcast_to(den[:, None, :], (r, KN, H)).reshape(x, H)
    attn = e / den_rep

    expand = (ri.T // DH == ci.T).astype(f32)
    wexp = jnp.dot(attn, expand, preferred_element_type=f32)
    outt = jnp.dot(seg, wexp * vg, preferred_element_type=f32)
    out_ref[...] = jnp.dot(outt, wout_ref[...], preferred_element_type=f32) + bout_ref[...]

    hin = jnp.dot(qk, cmw1_ref[...], preferred_element_type=f32) + cmb1_ref[...]
    hid = 0.5 * hin * (1.0 + lax.erf(hin * (2.0 ** -0.5)))
    cw = jnp.dot(hid, cmw2_ref[...], preferred_element_type=f32) + cmb2_ref[...]
    gate = jnp.tanh(jnp.dot(qk, cgw_ref[...], preferred_element_type=f32) + cgb_ref[...])
    reln = rel / jnp.maximum(dist, EPS) * cs_ref[0, 0]
    s = jnp.sum(cw * gate * cc_ref[...], axis=1, keepdims=True)
    delta = jnp.dot(seg, s * reln, preferred_element_type=f32)
    co_ref[...] = xi16[:, :3] + delta[:, :3]


def _attn(q, kvg, cgath, c16, w_out, b_out, cm_w1, cm_b1, cm_w2, cm_b2,
          cg_w, cg_b, cs, cc):
    bn = q.shape[0]
    r = R_ATT
    x = r * KN
    const = lambda shape: pl.BlockSpec(shape, lambda i: tuple(0 for _ in shape))
    return pl.pallas_call(
        _attn_body,
        grid=(bn // r,),
        in_specs=[
            pl.BlockSpec((r, 512), lambda i: (i, 0)),
            pl.BlockSpec((x, 512), lambda i: (i, 0)),
            pl.BlockSpec((x, 128), lambda i: (i, 0)),
            pl.BlockSpec((r, 16), lambda i: (i, 0)),
            const((512, 512)),
            const((1, 512)),
            const((H, 16)),
            const((1, 16)),
            const((16, H)),
            const((1, H)),
            const((H, H)),
            const((1, H)),
            const((1, 1)),
            const((1, H)),
        ],
        out_specs=[
            pl.BlockSpec((r, 512), lambda i: (i, 0)),
            pl.BlockSpec((r, 3), lambda i: (i, 0)),
        ],
        out_shape=[
            jax.ShapeDtypeStruct((bn, 512), jnp.float32),
            jax.ShapeDtypeStruct((bn, 3), jnp.float32),
        ],
    )(q, kvg, cgath, c16, w_out, b_out, cm_w1, cm_b1, cm_w2, cm_b2, cg_w, cg_b, cs, cc)


def kernel(feats, coors, W_qkv, W_out, b_out, cm_W1, cm_b1, cm_W2, cm_b2,
           cg_W, cg_b, coors_scale, coors_combine):
    b, n, d = feats.shape
    bn = b * n

    c16 = jnp.pad(coors.reshape(bn, 3), ((0, 0), (0, 13)))
    q, kvb, c128 = _qkv(feats.reshape(bn, d), W_qkv, c16)
    idx = _topk(coors, coors.transpose(0, 2, 1))
    kvg, cgath = _sc_gather(kvb, c128, idx.reshape(bn * KN))
    out, coors_out = _attn(
        q, kvg, cgath, c16, W_out, b_out.reshape(1, 512),
        cm_W1, cm_b1.reshape(1, 16), cm_W2, cm_b2.reshape(1, H),
        cg_W, cg_b.reshape(1, H), coors_scale.reshape(1, 1),
        coors_combine.reshape(1, H))
    return out.reshape(b, n, d), coors_out.reshape(b, n, 3)


# speedup vs baseline: 1.1439x; 1.0359x over previous
import functools
import math

import jax
import jax.numpy as jnp
from jax import lax
from jax.experimental import pallas as pl
from jax.experimental.pallas import tpu as pltpu
from jax.experimental.pallas import tpu_sc as plsc

H, DH, KN = 8, 64, 32
THETA = 10000.0
REL_DIST_CUTOFF = 5000.0
REL_DIST_SCALE = 100.0
EPS = 1e-8
SCALE = DH ** -0.5

R_TOPK = 256
R_ATT = 64
SC_CHUNK = 64


def _qkv_body(x_ref, w_ref, c_ref, q_ref, kvb_ref, c128_ref):
    y = jnp.dot(x_ref[...], w_ref[...], preferred_element_type=jnp.float32)
    q_ref[...] = y[:, :512]
    kb = y[:, 512:1024].astype(jnp.bfloat16).astype(jnp.float32)
    vb = y[:, 1024:].astype(jnp.bfloat16).astype(jnp.float32)
    ku = lax.shift_right_logical(lax.bitcast_convert_type(kb, jnp.uint32),
                                 jnp.uint32(16))
    vu = lax.bitcast_convert_type(vb, jnp.uint32) & jnp.uint32(0xFFFF0000)
    kvb_ref[...] = lax.bitcast_convert_type(ku | vu, jnp.int32)
    c128_ref[...] = jnp.pad(c_ref[...], ((0, 0), (0, 112)))


def _qkv(x, w, c16):
    bn, d = x.shape
    r = 256
    return pl.pallas_call(
        _qkv_body,
        grid=(bn // r,),
        in_specs=[
            pl.BlockSpec((r, d), lambda i: (i, 0)),
            pl.BlockSpec((d, 3 * 512), lambda i: (0, 0)),
            pl.BlockSpec((r, 16), lambda i: (i, 0)),
        ],
        out_specs=[
            pl.BlockSpec((r, 512), lambda i: (i, 0)),
            pl.BlockSpec((r, 512), lambda i: (i, 0)),
            pl.BlockSpec((r, 128), lambda i: (i, 0)),
        ],
        out_shape=[
            jax.ShapeDtypeStruct((bn, 512), jnp.float32),
            jax.ShapeDtypeStruct((bn, 512), jnp.int32),
            jax.ShapeDtypeStruct((bn, 128), jnp.float32),
        ],
    )(x, w, c16)


def _topk_body(n, xi_ref, xt_ref, idx_ref):
    b = pl.program_id(0)
    xi = xi_ref[0]
    xt = xt_ref[0]
    dsq = (xi[:, 0:1] - xt[0:1, :]) ** 2
    dsq += (xi[:, 1:2] - xt[1:2, :]) ** 2
    dsq += (xi[:, 2:3] - xt[2:3, :]) ** 2
    r = xi.shape[0]
    iota = lax.broadcasted_iota(jnp.int32, (r, n), 1)
    big = jnp.int32(n)
    inf = jnp.float32(jnp.inf)
    base = b * n
    for it in range(KN):
        m = jnp.min(dsq, axis=1, keepdims=True)
        tie = dsq == m
        a = jnp.min(jnp.where(tie, iota, big), axis=1, keepdims=True)
        idx_ref[0, :, it:it + 1] = a + base
        dsq = jnp.where(tie, inf, dsq)


def _topk(coors, coors_t):
    b, n, _ = coors.shape
    return pl.pallas_call(
        functools.partial(_topk_body, n),
        grid=(b, n // R_TOPK),
        in_specs=[
            pl.BlockSpec((1, R_TOPK, 3), lambda bi, i: (bi, i, 0)),
            pl.BlockSpec((1, 3, n), lambda bi, i: (bi, 0, 0)),
        ],
        out_specs=pl.BlockSpec((1, R_TOPK, KN), lambda bi, i: (bi, i, 0)),
        out_shape=jax.ShapeDtypeStruct((b, n, KN), jnp.int32),
    )(coors, coors_t)


def _sc_gather(kvb, c128, idx_flat):
    g = idx_flat.shape[0]
    ch = SC_CHUNK
    info = plsc.get_sparse_core_info()
    nw = info.num_cores * info.num_subcores
    per_w = g // nw
    n_chunks = per_w // ch
    mesh = plsc.VectorSubcoreMesh(core_axis_name="c", subcore_axis_name="s")

    @functools.partial(
        pl.kernel,
        mesh=mesh,
        out_type=[
            jax.ShapeDtypeStruct((g, 512), jnp.int32),
            jax.ShapeDtypeStruct((g, 128), jnp.float32),
        ],
        scratch_types=[
            pltpu.VMEM((ch,), jnp.int32),
            pltpu.VMEM((ch,), jnp.int32),
            pltpu.VMEM((ch, 512), jnp.int32),
            pltpu.VMEM((ch, 512), jnp.int32),
            pltpu.VMEM((ch, 128), jnp.float32),
            pltpu.VMEM((ch, 128), jnp.float32),
            pltpu.SemaphoreType.DMA,
            pltpu.SemaphoreType.DMA,
            pltpu.SemaphoreType.DMA,
            pltpu.SemaphoreType.DMA,
        ],
    )
    def k(kvb_hbm, c128_hbm, idx_hbm, kvg_hbm, cg_hbm,
          idx_a, idx_b, kv_a, kv_b, c_a, c_b, semka, semca, semkb, semcb):
        wid = lax.axis_index("s") * info.num_cores + lax.axis_index("c")
        wbase = wid * per_w
        bufs = ((idx_a, kv_a, c_a, semka, semca),
                (idx_b, kv_b, c_b, semkb, semcb))

        def fire(ci, bf):
            idx_v, kv_v, c_v, semk, semc = bf
            off = pl.multiple_of(wbase + ci * ch, ch)
            pltpu.sync_copy(idx_hbm.at[pl.ds(off, ch)], idx_v)
            pltpu.async_copy(kvb_hbm.at[idx_v], kv_v, semk)
            pltpu.async_copy(c128_hbm.at[idx_v], c_v, semc)

        def drain_store(ci, bf):
            idx_v, kv_v, c_v, semk, semc = bf
            off = pl.multiple_of(wbase + ci * ch, ch)
            pltpu.make_async_copy(kvb_hbm.at[pl.ds(0, ch)], kv_v, semk).wait()
            pltpu.make_async_copy(c128_hbm.at[pl.ds(0, ch)], c_v, semc).wait()
            pltpu.sync_copy(kv_v, kvg_hbm.at[pl.ds(off, ch)])
            pltpu.sync_copy(c_v, cg_hbm.at[pl.ds(off, ch)])

        fire(0, bufs[0])

        def body(p, carry):
            e = p * 2
            fire(e + 1, bufs[1])
            drain_store(e, bufs[0])

            @pl.when(e + 2 < n_chunks)
            def _():
                fire(e + 2, bufs[0])

            drain_store(e + 1, bufs[1])
            return carry

        lax.fori_loop(0, n_chunks // 2, body, 0)

    return k(kvb, c128, idx_flat)


def _attn_body(q_ref, kvg_ref, cg_ref, xi_ref, wout_ref, bout_ref,
               cmw1_ref, cmb1_ref, cmw2_ref, cmb2_ref, cgw_ref, cgb_ref,
               cs_ref, cc_ref, out_ref, co_ref):
    r = q_ref.shape[0]
    x = r * KN
    f32 = jnp.float32

    w = lax.bitcast_convert_type(kvg_ref[...], jnp.uint32)
    kg = lax.bitcast_convert_type(
        lax.shift_left(w, jnp.full(w.shape, 16, jnp.uint32)), f32)
    vg = lax.bitcast_convert_type(w & jnp.uint32(0xFFFF0000), f32)
    cg = cg_ref[:, :16]

    xi16 = xi_ref[...]
    xi_rep = jnp.broadcast_to(xi16[:, None, :], (r, KN, 16)).reshape(x, 16)
    rel = xi_rep - cg
    d2 = jnp.sum(rel * rel, axis=1, keepdims=True) + 1e-30
    dist = jnp.sqrt(d2)

    t = jnp.minimum(dist * REL_DIST_SCALE, REL_DIST_CUTOFF)
    pair = lax.broadcasted_iota(jnp.int32, (1, DH), 1) // 2
    inv_f = jnp.exp(pair.astype(f32) * (-math.log(THETA) * 2.0 / DH))
    ang = t * inv_f
    q = lax.round(ang * (1.0 / math.pi),
                  lax.RoundingMethod.TO_NEAREST_EVEN)
    rt = (ang - q * 3.140625) - q * 9.67653589793e-4
    z = rt * rt
    cosp = 1.0 + z * (-0.5 + z * (1.0 / 24 + z * (-1.0 / 720 + z * (
        1.0 / 40320 + z * (-1.0 / 3628800 + z * (1.0 / 479001600))))))
    sinp = rt * (1.0 + z * (-1.0 / 6 + z * (1.0 / 120 + z * (
        -1.0 / 5040 + z * (1.0 / 362880 + z * (-1.0 / 39916800))))))
    sgn = lax.shift_left(q.astype(jnp.int32), jnp.full(ang.shape, 31, jnp.int32))
    c64 = lax.bitcast_convert_type(
        lax.bitcast_convert_type(cosp, jnp.int32) ^ sgn, f32)
    s64 = lax.bitcast_convert_type(
        lax.bitcast_convert_type(sinp, jnp.int32) ^ sgn, f32)
    cosf = jnp.concatenate([c64] * H, axis=1)
    sinf = jnp.concatenate([s64] * H, axis=1)
    kl = jnp.concatenate([kg[:, 1:], kg[:, :1]], axis=1)
    kr = jnp.concatenate([kg[:, 511:], kg[:, :511]], axis=1)
    even = (lax.broadcasted_iota(jnp.int32, (1, 512), 1) % 2) == 0
    krot = jnp.where(even, -kl, kr)
    k_rot = kg * cosf + krot * sinf

    q = q_ref[...]
    q_rep = jnp.broadcast_to(q[:, None, :], (r, KN, 512)).reshape(x, 512)
    ri = lax.broadcasted_iota(jnp.int32, (512, H), 0)
    ci = lax.broadcasted_iota(jnp.int32, (512, H), 1)
    sumblk = (ri // DH == ci).astype(f32)
    qk = jnp.dot(q_rep * k_rot, sumblk, preferred_element_type=f32) * SCALE

    sri = lax.broadcasted_iota(jnp.int32, (r, x), 0)
    sci = lax.broadcasted_iota(jnp.int32, (r, x), 1)
    seg = (sci // KN == sri).astype(f32)

    e = jnp.exp(qk)
    den = jnp.dot(seg, e, preferred_element_type=f32)
    den_rep = jnp.broadcast_to(den[:, None, :], (r, KN, H)).reshape(x, H)
    attn = e / den_rep

    expand = (ri.T // DH == ci.T).astype(f32)
    wexp = jnp.dot(attn, expand, preferred_element_type=f32)
    outt = jnp.dot(seg, wexp * vg, preferred_element_type=f32)
    out_ref[...] = jnp.dot(outt, wout_ref[...], preferred_element_type=f32) + bout_ref[...]

    hin = jnp.dot(qk, cmw1_ref[...], preferred_element_type=f32) + cmb1_ref[...]
    hid = 0.5 * hin * (1.0 + lax.erf(hin * (2.0 ** -0.5)))
    cw = jnp.dot(hid, cmw2_ref[...], preferred_element_type=f32) + cmb2_ref[...]
    gate = jnp.tanh(jnp.dot(qk, cgw_ref[...], preferred_element_type=f32) + cgb_ref[...])
    reln = rel / jnp.maximum(dist, EPS) * cs_ref[0, 0]
    s = jnp.sum(cw * gate * cc_ref[...], axis=1, keepdims=True)
    delta = jnp.dot(seg, s * reln, preferred_element_type=f32)
    co_ref[...] = xi16[:, :3] + delta[:, :3]


def _attn(q, kvg, cgath, c16, w_out, b_out, cm_w1, cm_b1, cm_w2, cm_b2,
          cg_w, cg_b, cs, cc):
    bn = q.shape[0]
    r = R_ATT
    x = r * KN
    const = lambda shape: pl.BlockSpec(shape, lambda i: tuple(0 for _ in shape))
    return pl.pallas_call(
        _attn_body,
        grid=(bn // r,),
        in_specs=[
            pl.BlockSpec((r, 512), lambda i: (i, 0)),
            pl.BlockSpec((x, 512), lambda i: (i, 0)),
            pl.BlockSpec((x, 128), lambda i: (i, 0)),
            pl.BlockSpec((r, 16), lambda i: (i, 0)),
            const((512, 512)),
            const((1, 512)),
            const((H, 16)),
            const((1, 16)),
            const((16, H)),
            const((1, H)),
            const((H, H)),
            const((1, H)),
            const((1, 1)),
            const((1, H)),
        ],
        out_specs=[
            pl.BlockSpec((r, 512), lambda i: (i, 0)),
            pl.BlockSpec((r, 3), lambda i: (i, 0)),
        ],
        out_shape=[
            jax.ShapeDtypeStruct((bn, 512), jnp.float32),
            jax.ShapeDtypeStruct((bn, 3), jnp.float32),
        ],
    )(q, kvg, cgath, c16, w_out, b_out, cm_w1, cm_b1, cm_w2, cm_b2, cg_w, cg_b, cs, cc)


def kernel(feats, coors, W_qkv, W_out, b_out, cm_W1, cm_b1, cm_W2, cm_b2,
           cg_W, cg_b, coors_scale, coors_combine):
    b, n, d = feats.shape
    bn = b * n

    c16 = jnp.pad(coors.reshape(bn, 3), ((0, 0), (0, 13)))
    q, kvb, c128 = _qkv(feats.reshape(bn, d), W_qkv, c16)
    idx = _topk(coors, coors.transpose(0, 2, 1))
    kvg, cgath = _sc_gather(kvb, c128, idx.reshape(bn * KN))
    out, coors_out = _attn(
        q, kvg, cgath, c16, W_out, b_out.reshape(1, 512),
        cm_W1, cm_b1.reshape(1, 16), cm_W2, cm_b2.reshape(1, H),
        cg_W, cg_b.reshape(1, H), coors_scale.reshape(1, 1),
        coors_combine.reshape(1, H))
    return out.reshape(b, n, d), coors_out.reshape(b, n, 3)
